# baseline (speedup 1.0000x reference)
import jax
import jax.numpy as jnp
from jax import lax
from jax.experimental import pallas as pl
from jax.experimental.pallas import tpu as pltpu
from jax.experimental.pallas import tpu_sc as plsc

B = 16384
D = 16
L = 50
NE = 8
F = (NE + 1) * D
NC = 2
NS = 16
NW = NC * NS
S = B // NW
C = 128
NCH = S // C
NG = C // 16

HIST_COL = NE * D
IOTA = None


def _sc_gather_body(idx8_hbm, hist_hbm,
                    emb_user, emb_item, ec0, ec1, ec2, ec3, ec4, ec5,
                    emb_hist,
                    feats_hbm,
                    i8s_v, hs_v, hi8_v, off8_v, hhi_v, hoff_v,
                    big_v, feats_v,
                    sem0, sem1, semi):
    tables = (emb_user, emb_item, ec0, ec1, ec2, ec3, ec4, ec5)
    wid = lax.axis_index("s") * NC + lax.axis_index("c")
    lane = jax.lax.broadcasted_iota(jnp.int32, (16,), 0)

    def fire(table, idx_ref, buf, sem):
        return pltpu.async_copy(table.at[idx_ref], big_v.at[buf], sem)

    def select_group(buf, g, t_col, off_ref_row, accumulate, kpiece=None):
        rows = g * 16 + lane
        off16 = off_ref_row[pl.ds(g * 16, 16)]
        colbase = off16 * 16
        for d in range(D):
            v = plsc.load_gather(big_v.at[buf], [rows, colbase + d])
            cols = jnp.full((16,), t_col + d, jnp.int32)
            if accumulate:
                plsc.addupdate_scatter(feats_v, [rows, cols], v)
            else:
                plsc.store_scatter(feats_v, [rows, cols], v)

    @pl.loop(0, NCH)
    def _chunk(c):
        base = wid * S + c * C

        pltpu.sync_copy(idx8_hbm.at[:, pl.ds(base, C)], i8s_v)
        pltpu.sync_copy(hist_hbm.at[:, pl.ds(base, C)], hs_v)

        for t in range(NE):
            @pl.loop(0, NG)
            def _split8(g, t=t):
                raw = i8s_v[t, pl.ds(g * 16, 16)]
                hi8_v[t, pl.ds(g * 16, 16)] = raw >> 3
                off8_v[t, pl.ds(g * 16, 16)] = raw & 7

        @pl.loop(0, L)
        def _splith(k):
            @pl.loop(0, NG)
            def _splithg(g):
                raw = hs_v[k, pl.ds(g * 16, 16)]
                hhi_v[k, pl.ds(g * 16, 16)] = raw >> 3
                hoff_v[k, pl.ds(g * 16, 16)] = raw & 7

        fire(tables[0], hi8_v.at[0], 0, sem0)
        fire(tables[1], hi8_v.at[1], 1, sem1)
        for t in range(NE):
            sem = sem0 if t % 2 == 0 else sem1
            pltpu.make_async_copy(tables[t].at[hi8_v.at[t]],
                                  big_v.at[t % 2], sem).wait()
            if t + 2 < NE:
                fire(tables[t + 2], hi8_v.at[t + 2], t % 2, sem)
            elif t + 2 == NE:
                fire(emb_hist, hhi_v.at[0], 0, sem0)
            elif t + 2 == NE + 1:
                fire(emb_hist, hhi_v.at[1], 1, sem1)

            @pl.loop(0, NG)
            def _selg(g, t=t):
                select_group(t % 2, g, t * D, off8_v.at[t], False)

        pltpu.make_async_copy(emb_hist.at[hhi_v.at[0]], big_v.at[0],
                              sem0).wait()
        fire(emb_hist, hhi_v.at[2], 0, sem0)

        @pl.loop(0, NG)
        def _h0(g):
            select_group(0, g, HIST_COL, hoff_v.at[0], False)

        @pl.loop(1, L - 1, step=2)
        def _hist(k):
            pltpu.make_async_copy(emb_hist.at[hhi_v.at[k]], big_v.at[1],
                                  sem1).wait()

            @pl.loop(0, NG)
            def _hg1(g):
                select_group(1, g, HIST_COL, hoff_v.at[k], True)

            @pl.when(k + 2 < L)
            def _f1():
                fire(emb_hist, hhi_v.at[k + 2], 1, sem1)

            pltpu.make_async_copy(emb_hist.at[hhi_v.at[k + 1]],
                                  big_v.at[0], sem0).wait()

            @pl.loop(0, NG)
            def _hg0(g):
                select_group(0, g, HIST_COL, hoff_v.at[k + 1], True)

            @pl.when(k + 3 < L)
            def _f0():
                fire(emb_hist, hhi_v.at[k + 3], 0, sem0)

        pltpu.make_async_copy(emb_hist.at[hhi_v.at[L - 1]], big_v.at[1],
                              sem1).wait()

        @pl.loop(0, NG)
        def _hlast(g):
            select_group(1, g, HIST_COL, hoff_v.at[L - 1], True)

        pltpu.sync_copy(feats_v, feats_hbm.at[pl.ds(base, C), :])


def _sc_gather(idx8, histT, emb_user, emb_item, ec0, ec1, ec2, ec3, ec4,
               ec5, emb_hist):
    mesh = plsc.VectorSubcoreMesh(core_axis_name="c", subcore_axis_name="s")
    return pl.kernel(
        _sc_gather_body,
        out_type=jax.ShapeDtypeStruct((B, F), jnp.float32),
        mesh=mesh,
        scratch_types=[
            pltpu.VMEM((NE, C), jnp.int32),
            pltpu.VMEM((L, C), jnp.int32),
            pltpu.VMEM((NE, C), jnp.int32),
            pltpu.VMEM((NE, C), jnp.int32),
            pltpu.VMEM((L, C), jnp.int32),
            pltpu.VMEM((L, C), jnp.int32),
            pltpu.VMEM((2, C, 128), jnp.float32),
            pltpu.VMEM((C, F), jnp.float32),
            pltpu.SemaphoreType.DMA,
            pltpu.SemaphoreType.DMA,
            pltpu.SemaphoreType.DMA,
        ],
        compiler_params=pltpu.CompilerParams(use_tc_tiling_on_sc=True,
                                             needs_layout_passes=False),
    )(idx8, histT, emb_user, emb_item, ec0, ec1, ec2, ec3, ec4, ec5,
      emb_hist)


def _mlp_body(x_ref, ctn_ref, wv_ref, W1_ref, b1_ref, W2_ref, b2_ref,
              W3_ref, b3_ref, o_ref):
    x = x_ref[...]
    h = jnp.maximum(x @ W1_ref[...] + b1_ref[...][None, :], 0.0)
    h = jnp.maximum(h @ W2_ref[...] + b2_ref[...][None, :], 0.0)
    z = h @ W3_ref[...]
    lin = ctn_ref[...] @ wv_ref[...]
    r = z[:, 0] + lin[:, 0] + b3_ref[0]
    o_ref[...] = jax.nn.sigmoid(r)


def _mlp(feats, ctn, wvec, W1, b1, W2, b2, W3, b3):
    bm = 2048
    grid = (B // bm,)
    return pl.pallas_call(
        _mlp_body,
        grid=grid,
        in_specs=[
            pl.BlockSpec((bm, F), lambda i: (i, 0)),
            pl.BlockSpec((bm, 4), lambda i: (i, 0)),
            pl.BlockSpec((4, 1), lambda i: (0, 0)),
            pl.BlockSpec((F, 256), lambda i: (0, 0)),
            pl.BlockSpec((256,), lambda i: (0,)),
            pl.BlockSpec((256, 128), lambda i: (0, 0)),
            pl.BlockSpec((128,), lambda i: (0,)),
            pl.BlockSpec((128, 1), lambda i: (0, 0)),
            pl.BlockSpec((1,), lambda i: (0,)),
        ],
        out_specs=pl.BlockSpec((bm,), lambda i: (i,)),
        out_shape=jax.ShapeDtypeStruct((B,), jnp.float32),
    )(feats, ctn, wvec, W1, b1, W2, b2, W3, b3)


def kernel(user_id, item_id, cat_0, cat_1, cat_2, cat_3, cat_4, cat_5,
           ctn_0, ctn_1, ctn_2, ctn_3, hist_item,
           emb_user, emb_item, emb_cat_0, emb_cat_1, emb_cat_2, emb_cat_3,
           emb_cat_4, emb_cat_5, emb_hist,
           w_ctn_0, w_ctn_1, w_ctn_2, w_ctn_3,
           W1, b1, W2, b2, W3, b3):
    idx8 = jnp.stack([
        user_id[:, 0], item_id[:, 0], cat_0[:, 0], cat_1[:, 0],
        cat_2[:, 0], cat_3[:, 0], cat_4[:, 0], cat_5[:, 0],
    ]).astype(jnp.int32)
    histT = hist_item.T.astype(jnp.int32)

    r8 = lambda t: t.reshape(t.shape[0] // 8, t.shape[1] * 8)
    feats = _sc_gather(idx8, histT, r8(emb_user), r8(emb_item),
                       r8(emb_cat_0), r8(emb_cat_1), r8(emb_cat_2),
                       r8(emb_cat_3), r8(emb_cat_4), r8(emb_cat_5),
                       r8(emb_hist))

    ctn = jnp.concatenate([ctn_0, ctn_1, ctn_2, ctn_3], axis=1)
    wvec = jnp.stack([w_ctn_0[0, 0], w_ctn_1[0, 0], w_ctn_2[0, 0],
                      w_ctn_3[0, 0]]).reshape(4, 1)
    return _mlp(feats, ctn, wvec, W1, b1, W2, b2, W3, b3)
